# Initial kernel scaffold; baseline (speedup 1.0000x reference)
import jax
import jax.numpy as jnp
from jax.experimental import pallas as pl


def kernel(feat, edge_index, etypes, W_fwd, W_bwd, forward_bias, backward_bias, self_bias, loop_weight, use_dis_weight, drop_out):
    raise NotImplementedError("write your pallas kernel here")


# speedup vs baseline: 9.3613x; 9.3613x over previous
import functools

import jax
import jax.numpy as jnp
from jax import lax
from jax.experimental import pallas as pl
from jax.experimental.pallas import tpu as pltpu
from jax.experimental.pallas import tpu_sc as plsc

N = 10000
E = 160000
R = 4
F = 128

NC = 2
NS = 16
EDGES_PER_SUB = E // NS
CHUNK = 80
NCHUNK = EDGES_PER_SUB // CHUNK
ROWS_PER_SUB = N // NS
DEG_W = 16

BN = 2000


def _tables_body(feat_ref, wf_ref, wb_ref, lw_ref, hf_ref, hb_ref, sl_ref):
    x = feat_ref[...]
    hf_ref[...] = jnp.dot(x, wf_ref[...], preferred_element_type=jnp.float32)
    hb_ref[...] = jnp.dot(x, wb_ref[...], preferred_element_type=jnp.float32)
    sl = jnp.dot(x, lw_ref[...], preferred_element_type=jnp.float32)
    sl_ref[...] = sl + sl


def _tables(feat, wf_cat, wb_cat, loop_weight):
    grid = (N // BN,)
    return pl.pallas_call(
        _tables_body,
        grid=grid,
        in_specs=[
            pl.BlockSpec((BN, F), lambda i: (i, 0)),
            pl.BlockSpec((F, R * F), lambda i: (0, 0)),
            pl.BlockSpec((F, R * F), lambda i: (0, 0)),
            pl.BlockSpec((F, F), lambda i: (0, 0)),
        ],
        out_specs=[
            pl.BlockSpec((BN, R * F), lambda i: (i, 0)),
            pl.BlockSpec((BN, R * F), lambda i: (i, 0)),
            pl.BlockSpec((BN, F), lambda i: (i, 0)),
        ],
        out_shape=[
            jax.ShapeDtypeStruct((N, R * F), jnp.float32),
            jax.ShapeDtypeStruct((N, R * F), jnp.float32),
            jax.ShapeDtypeStruct((N, F), jnp.float32),
        ],
    )(feat, wf_cat, wb_cat, loop_weight)


WB = 624
WB_TAIL = N - NS * WB


def _sliced_copy(src, dst, s, add=False):
    pltpu.sync_copy(src.at[pl.ds(s * WB, WB)], dst.at[pl.ds(s * WB, WB)],
                    add=add)

    @pl.when(s == 0)
    def _():
        pltpu.sync_copy(src.at[pl.ds(NS * WB, WB_TAIL)],
                        dst.at[pl.ds(NS * WB, WB_TAIL)], add=add)


def _sc_body(hf, hb, idxf, dstf, idxb, dstb, zacc, ones_h,
             accf, accb, degf, degb,
             acc_s, idx_v, didx_v, rows_v, ones_v, sem):
    c = lax.axis_index("c")
    s = lax.axis_index("s")

    _sliced_copy(zacc, acc_s, s)
    pltpu.sync_copy(ones_h, ones_v)
    plsc.subcore_barrier()

    ebase = s * EDGES_PER_SUB

    def gather_scatter(tab, idx_h, didx_h):
        def step(j, carry):
            off = ebase + j * CHUNK
            pltpu.sync_copy(idx_h.at[pl.ds(off, CHUNK)], idx_v)
            pltpu.sync_copy(didx_h.at[pl.ds(off, CHUNK)], didx_v)
            pltpu.async_copy(tab.at[idx_v], rows_v, sem).wait()
            pltpu.sync_copy(rows_v, acc_s.at[didx_v], add=True)
            return carry

        lax.fori_loop(0, NCHUNK, step, 0)

    def count_deg(didx_h):
        def step(j, carry):
            off = ebase + j * CHUNK
            pltpu.sync_copy(didx_h.at[pl.ds(off, CHUNK)], didx_v)
            pltpu.sync_copy(ones_v, acc_s.at[didx_v], add=True)
            return carry

        lax.fori_loop(0, NCHUNK, step, 0)

    @pl.when(c == 0)
    def _():
        gather_scatter(hf, idxf, dstf)

    @pl.when(c == 1)
    def _():
        gather_scatter(hb, idxb, dstb)

    plsc.subcore_barrier()

    @pl.when(c == 0)
    def _():
        _sliced_copy(acc_s, accf, s)

    @pl.when(c == 1)
    def _():
        _sliced_copy(acc_s, accb, s)

    plsc.subcore_barrier()
    _sliced_copy(zacc, acc_s, s)
    plsc.subcore_barrier()

    @pl.when(c == 0)
    def _():
        count_deg(dstf)

    @pl.when(c == 1)
    def _():
        count_deg(dstb)

    plsc.subcore_barrier()

    @pl.when(c == 0)
    def _():
        _sliced_copy(acc_s, degf, s)

    @pl.when(c == 1)
    def _():
        _sliced_copy(acc_s, degb, s)


_sc_call = functools.partial(
    pl.kernel,
    out_type=(
        jax.ShapeDtypeStruct((N, F), jnp.float32),
        jax.ShapeDtypeStruct((N, F), jnp.float32),
        jax.ShapeDtypeStruct((N, F), jnp.float32),
        jax.ShapeDtypeStruct((N, F), jnp.float32),
    ),
    mesh=plsc.VectorSubcoreMesh(core_axis_name="c", subcore_axis_name="s",
                                num_cores=NC, num_subcores=NS),
    scratch_types=[
        pltpu.VMEM_SHARED((N, F), jnp.float32),
        pltpu.VMEM((CHUNK,), jnp.int32),
        pltpu.VMEM((CHUNK,), jnp.int32),
        pltpu.VMEM((CHUNK, F), jnp.float32),
        pltpu.VMEM((CHUNK, F), jnp.float32),
        pltpu.SemaphoreType.DMA,
    ],
)(_sc_body)


def _combine_body(sl_ref, accf_ref, accb_ref, degf_ref, degb_ref,
                  fb_ref, bb_ref, sb_ref, out_ref):
    df = degf_ref[:, 0:1]
    db = degb_ref[:, 0:1]
    invf = jnp.where(df > 0.0, 1.0 / jnp.maximum(df, 1.0), 0.0)
    invb = jnp.where(db > 0.0, 1.0 / jnp.maximum(db, 1.0), 0.0)
    gatef = jnp.where(df > 0.0, 1.0, 0.0)
    gateb = jnp.where(db > 0.0, 1.0, 0.0)
    out_ref[...] = (sl_ref[...] + sb_ref[...]
                    + accf_ref[...] * invf + gatef * fb_ref[...]
                    + accb_ref[...] * invb + gateb * bb_ref[...])


def _combine(sl, accf, accb, degf, degb, fb, bb, sb):
    grid = (N // BN,)
    return pl.pallas_call(
        _combine_body,
        grid=grid,
        in_specs=[
            pl.BlockSpec((BN, F), lambda i: (i, 0)),
            pl.BlockSpec((BN, F), lambda i: (i, 0)),
            pl.BlockSpec((BN, F), lambda i: (i, 0)),
            pl.BlockSpec((BN, F), lambda i: (i, 0)),
            pl.BlockSpec((BN, F), lambda i: (i, 0)),
            pl.BlockSpec((1, F), lambda i: (0, 0)),
            pl.BlockSpec((1, F), lambda i: (0, 0)),
            pl.BlockSpec((1, F), lambda i: (0, 0)),
        ],
        out_specs=pl.BlockSpec((BN, F), lambda i: (i, 0)),
        out_shape=jax.ShapeDtypeStruct((N, F), jnp.float32),
    )(sl, accf, accb, degf, degb, fb, bb, sb)


def kernel(feat, edge_index, etypes, W_fwd, W_bwd, forward_bias,
           backward_bias, self_bias, loop_weight, use_dis_weight, drop_out):
    src = edge_index[0].astype(jnp.int32)
    dst = edge_index[1].astype(jnp.int32)
    et = etypes.astype(jnp.int32)

    idx_f = src * R + et
    dst_f = dst
    idx_b = dst * R + et
    dst_b = src

    wf_cat = W_fwd.transpose(1, 0, 2).reshape(F, R * F)
    wb_cat = W_bwd.transpose(1, 0, 2).reshape(F, R * F)

    hf, hb, sl = _tables(feat, wf_cat, wb_cat, loop_weight)
    hf = hf.reshape(N * R, F)
    hb = hb.reshape(N * R, F)

    zacc = jnp.zeros((N, F), jnp.float32)
    ones_h = jnp.ones((CHUNK, F), jnp.float32)

    accf, accb, degf, degb = _sc_call(hf, hb, idx_f, dst_f, idx_b, dst_b,
                                      zacc, ones_h)

    return _combine(sl, accf, accb, degf, degb,
                    forward_bias.reshape(1, F), backward_bias.reshape(1, F),
                    self_bias.reshape(1, F))


# speedup vs baseline: 11.7540x; 1.2556x over previous
import functools

import jax
import jax.numpy as jnp
from jax import lax
from jax.experimental import pallas as pl
from jax.experimental.pallas import tpu as pltpu
from jax.experimental.pallas import tpu_sc as plsc

N = 10000
E = 160000
R = 4
F = 128

NC = 2
NS = 16
CHUNK = 128
NCHUNK = 79
EPS = NCHUNK * CHUNK
EPAD = NS * EPS
NPAD = N + 8

BN = 2000


def _tables_body(feat_ref, wf_ref, wb_ref, lw_ref, hf_ref, hb_ref, sl_ref):
    x = feat_ref[...]
    hf_ref[...] = jnp.dot(x, wf_ref[...], preferred_element_type=jnp.float32)
    hb_ref[...] = jnp.dot(x, wb_ref[...], preferred_element_type=jnp.float32)
    sl = jnp.dot(x, lw_ref[...], preferred_element_type=jnp.float32)
    sl_ref[...] = sl + sl


def _tables(feat, wf_cat, wb_cat, loop_weight):
    grid = (N // BN,)
    return pl.pallas_call(
        _tables_body,
        grid=grid,
        in_specs=[
            pl.BlockSpec((BN, F), lambda i: (i, 0)),
            pl.BlockSpec((F, R * F), lambda i: (0, 0)),
            pl.BlockSpec((F, R * F), lambda i: (0, 0)),
            pl.BlockSpec((F, F), lambda i: (0, 0)),
        ],
        out_specs=[
            pl.BlockSpec((BN, R * F), lambda i: (i, 0)),
            pl.BlockSpec((BN, R * F), lambda i: (i, 0)),
            pl.BlockSpec((BN, F), lambda i: (i, 0)),
        ],
        out_shape=[
            jax.ShapeDtypeStruct((N, R * F), jnp.float32),
            jax.ShapeDtypeStruct((N, R * F), jnp.float32),
            jax.ShapeDtypeStruct((N, F), jnp.float32),
        ],
    )(feat, wf_cat, wb_cat, loop_weight)


WB = 624
WB_TAIL = NPAD - NS * WB


def _sliced_copy(src, dst, s, add=False):
    pltpu.sync_copy(src.at[pl.ds(s * WB, WB)], dst.at[pl.ds(s * WB, WB)],
                    add=add)

    @pl.when(s == 0)
    def _():
        pltpu.sync_copy(src.at[pl.ds(NS * WB, WB_TAIL)],
                        dst.at[pl.ds(NS * WB, WB_TAIL)], add=add)


def _sc_body(hf, hb, idxf, dstf, idxb, dstb, zacc, ones_h,
             accf, accb, degf, degb,
             acc_s, idx_all, didx_all, rows_v, sem):
    c = lax.axis_index("c")
    s = lax.axis_index("s")

    _sliced_copy(zacc, acc_s, s)
    plsc.subcore_barrier()

    @pl.when(c == 0)
    def _():
        pltpu.sync_copy(idxf.at[s], idx_all)
        pltpu.sync_copy(dstf.at[s], didx_all)

    @pl.when(c == 1)
    def _():
        pltpu.sync_copy(idxb.at[s], idx_all)
        pltpu.sync_copy(dstb.at[s], didx_all)

    def gather_scatter(tab):
        def step(j, carry):
            pltpu.async_copy(tab.at[idx_all.at[j]], rows_v, sem).wait()
            pltpu.sync_copy(rows_v, acc_s.at[didx_all.at[j]], add=True)
            return carry

        lax.fori_loop(0, NCHUNK, step, 0)

    @pl.when(c == 0)
    def _():
        gather_scatter(hf)

    @pl.when(c == 1)
    def _():
        gather_scatter(hb)

    plsc.subcore_barrier()

    @pl.when(c == 0)
    def _():
        _sliced_copy(acc_s, accf, s)

    @pl.when(c == 1)
    def _():
        _sliced_copy(acc_s, accb, s)

    plsc.subcore_barrier()
    _sliced_copy(zacc, acc_s, s)
    pltpu.sync_copy(ones_h, rows_v)
    plsc.subcore_barrier()

    def count_deg():
        def step(j, carry):
            pltpu.sync_copy(rows_v, acc_s.at[didx_all.at[j]], add=True)
            return carry

        lax.fori_loop(0, NCHUNK, step, 0)

    count_deg()
    plsc.subcore_barrier()

    @pl.when(c == 0)
    def _():
        _sliced_copy(acc_s, degf, s)

    @pl.when(c == 1)
    def _():
        _sliced_copy(acc_s, degb, s)


_sc_call = functools.partial(
    pl.kernel,
    out_type=(
        jax.ShapeDtypeStruct((NPAD, F), jnp.float32),
        jax.ShapeDtypeStruct((NPAD, F), jnp.float32),
        jax.ShapeDtypeStruct((NPAD, F), jnp.float32),
        jax.ShapeDtypeStruct((NPAD, F), jnp.float32),
    ),
    mesh=plsc.VectorSubcoreMesh(core_axis_name="c", subcore_axis_name="s",
                                num_cores=NC, num_subcores=NS),
    scratch_types=[
        pltpu.VMEM_SHARED((NPAD, F), jnp.float32),
        pltpu.VMEM((NCHUNK, CHUNK), jnp.int32),
        pltpu.VMEM((NCHUNK, CHUNK), jnp.int32),
        pltpu.VMEM((CHUNK, F), jnp.float32),
        pltpu.SemaphoreType.DMA,
    ],
)(_sc_body)


def _combine_body(sl_ref, accf_ref, accb_ref, degf_ref, degb_ref,
                  fb_ref, bb_ref, sb_ref, out_ref):
    df = degf_ref[:, 0:1]
    db = degb_ref[:, 0:1]
    invf = jnp.where(df > 0.0, 1.0 / jnp.maximum(df, 1.0), 0.0)
    invb = jnp.where(db > 0.0, 1.0 / jnp.maximum(db, 1.0), 0.0)
    gatef = jnp.where(df > 0.0, 1.0, 0.0)
    gateb = jnp.where(db > 0.0, 1.0, 0.0)
    out_ref[...] = (sl_ref[...] + sb_ref[...]
                    + accf_ref[...] * invf + gatef * fb_ref[...]
                    + accb_ref[...] * invb + gateb * bb_ref[...])


def _combine(sl, accf, accb, degf, degb, fb, bb, sb):
    grid = (N // BN,)
    return pl.pallas_call(
        _combine_body,
        grid=grid,
        in_specs=[
            pl.BlockSpec((BN, F), lambda i: (i, 0)),
            pl.BlockSpec((BN, F), lambda i: (i, 0)),
            pl.BlockSpec((BN, F), lambda i: (i, 0)),
            pl.BlockSpec((BN, F), lambda i: (i, 0)),
            pl.BlockSpec((BN, F), lambda i: (i, 0)),
            pl.BlockSpec((1, F), lambda i: (0, 0)),
            pl.BlockSpec((1, F), lambda i: (0, 0)),
            pl.BlockSpec((1, F), lambda i: (0, 0)),
        ],
        out_specs=pl.BlockSpec((BN, F), lambda i: (i, 0)),
        out_shape=jax.ShapeDtypeStruct((N, F), jnp.float32),
    )(sl, accf, accb, degf, degb, fb, bb, sb)


def kernel(feat, edge_index, etypes, W_fwd, W_bwd, forward_bias,
           backward_bias, self_bias, loop_weight, use_dis_weight, drop_out):
    src = edge_index[0].astype(jnp.int32)
    dst = edge_index[1].astype(jnp.int32)
    et = etypes.astype(jnp.int32)

    def pad3(a, fill):
        return jnp.concatenate(
            [a, jnp.full((EPAD - E,), fill, jnp.int32)]).reshape(
                NS, NCHUNK, CHUNK)

    idx_f = pad3(src * R + et, 0)
    dst_f = pad3(dst, N)
    idx_b = pad3(dst * R + et, 0)
    dst_b = pad3(src, N)

    wf_cat = W_fwd.transpose(1, 0, 2).reshape(F, R * F)
    wb_cat = W_bwd.transpose(1, 0, 2).reshape(F, R * F)

    hf, hb, sl = _tables(feat, wf_cat, wb_cat, loop_weight)
    hf = hf.reshape(N * R, F)
    hb = hb.reshape(N * R, F)

    zacc = jnp.zeros((NPAD, F), jnp.float32)
    ones_h = jnp.ones((CHUNK, F), jnp.float32)

    accf, accb, degf, degb = _sc_call(hf, hb, idx_f, dst_f, idx_b, dst_b,
                                      zacc, ones_h)

    return _combine(sl, accf, accb, degf, degb,
                    forward_bias.reshape(1, F), backward_bias.reshape(1, F),
                    self_bias.reshape(1, F))
